# baseline (speedup 1.0000x reference)
import functools

import jax
import jax.numpy as jnp
from jax.experimental import pallas as pl

_K = 16
_BN = 256
_CUTOFF = 3.0
_RMIN = 3.5
_ETA = 4.0
_NUM_RADIAL = 5
_NUM_3BODY = 12


def _pairnn_block(x_ref, y_ref, z_ref, w1t_ref, w1_ref, w2t_ref, w2_ref,
                  w3_ref, b1_ref, b2_ref, b3_ref,
                  eacc_ref, gx_ref, gy_ref, gz_ref):
    f32 = jnp.float32
    bf16 = jnp.bfloat16
    K, BN = _K, _BN
    BP = K * BN

    def b32(a):
        return a.astype(bf16).astype(f32)

    x = x_ref[...]
    y = y_ref[...]
    z = z_ref[...]

    r2 = x * x + y * y + z * z
    r = jnp.sqrt(r2)
    rm = jnp.maximum(r, 1e-12)
    inv = 1.0 / rm
    ux = x * inv
    uy = y * inv
    uz = z * inv

    aa = jnp.pi / (_CUTOFF - _RMIN)
    arg = aa * (r - _RMIN)
    in_tail = r > _RMIN
    fc = jnp.where(in_tail, 0.5 + 0.5 * jnp.cos(arg), 1.0)
    dfc = jnp.where(in_tail, -0.5 * aa * jnp.sin(arg), 0.0)

    bk = jnp.pi / _CUTOFF
    fck = 0.5 + 0.5 * jnp.cos(bk * r)
    dfck = -0.5 * bk * jnp.sin(bk * r)

    c0 = jnp.sqrt(2.0 / _CUTOFF)
    an = (bk * (1.0 + jnp.arange(_NUM_RADIAL, dtype=jnp.int32)
                .astype(f32))).reshape(_NUM_RADIAL, 1, 1)
    s5 = jnp.sin(an * r[None])
    c5 = jnp.cos(an * r[None])
    rbf = (c0 * fc * inv)[None] * s5

    uxb = b32(ux)
    uyb = b32(uy)
    uzb = b32(uz)
    eyemask = (1.0 - jnp.eye(K, dtype=f32))[:, :, None]
    gram = (uxb[:, None, :] * uxb[None, :, :] +
            uyb[:, None, :] * uyb[None, :, :] +
            uzb[:, None, :] * uzb[None, :, :]) * eyemask

    mu = (-1.0 + (2.0 / (_NUM_3BODY - 1)) *
          jnp.arange(_NUM_3BODY, dtype=jnp.int32).astype(f32))
    mu4 = mu.reshape(1, _NUM_3BODY, 1, 1)
    cm4 = jnp.exp(-_ETA * mu * mu).reshape(1, _NUM_3BODY, 1, 1)
    ga = (-_ETA) * gram * gram
    gb2 = (2.0 * _ETA) * gram
    e4 = jnp.exp(ga[:, None] + mu4 * gb2[:, None]) * cm4
    g3b = jnp.sum(e4 * fck[:, None, None, :], axis=0)

    desc = jnp.concatenate([rbf, g3b], axis=0).reshape(
        _NUM_RADIAL + _NUM_3BODY, BP)

    def dot(a, b):
        return jax.lax.dot_general(
            a.astype(bf16), b.astype(bf16),
            dimension_numbers=(((1,), (0,)), ((), ())),
            preferred_element_type=f32)

    h1p = dot(w1t_ref[...], desc) + b1_ref[...]
    h1 = jnp.maximum(h1p, 0.0)
    h2p = dot(w2t_ref[...], h1) + b2_ref[...]
    h2 = jnp.maximum(h2p, 0.0)
    pre3 = (jnp.sum(b32(h2) * b32(w3_ref[...]), axis=0, keepdims=True)
            + b3_ref[...])
    pre3 = pre3.reshape(K, BN)

    eij = pre3 * fc
    part = eij[:8, :] + eij[8:, :]

    @pl.when(pl.program_id(0) == 0)
    def _():
        eacc_ref[...] = jnp.zeros_like(eacc_ref)

    eacc_ref[...] += part

    gfc = pre3
    gpre3 = b32(fc).reshape(1, BP)
    gh2 = b32(w3_ref[...]) * gpre3
    gh2p = jnp.where(h2p > 0.0, gh2, 0.0)
    gh1 = dot(w2_ref[...], gh2p)
    gh1p = jnp.where(h1p > 0.0, gh1, 0.0)
    gdesc = dot(w1_ref[...], gh1p).reshape(
        _NUM_RADIAL + _NUM_3BODY, K, BN)

    grbf = gdesc[:_NUM_RADIAL]
    gd3b = gdesc[_NUM_RADIAL:]

    gfc = gfc + c0 * jnp.sum(grbf * s5, axis=0) * inv
    gr = (c0 * fc * inv) * jnp.sum(grbf * (an * c5 - s5 * inv[None]), axis=0)

    s0 = jnp.sum(e4 * gd3b[None], axis=1)
    s1 = jnp.sum(e4 * (gd3b * mu.reshape(-1, 1, 1))[None], axis=1)
    gfckv = jnp.sum(s0, axis=1)
    ggram = (-2.0 * _ETA) * (gram * s0 - s1) * \
        fck[:, None, :] * eyemask

    ggb = b32(ggram)
    gdx = (jnp.sum(ggb * uxb[:, None, :], axis=0) +
           jnp.sum(ggb * uxb[None, :, :], axis=1))
    gdy = (jnp.sum(ggb * uyb[:, None, :], axis=0) +
           jnp.sum(ggb * uyb[None, :, :], axis=1))
    gdz = (jnp.sum(ggb * uzb[:, None, :], axis=0) +
           jnp.sum(ggb * uzb[None, :, :], axis=1))

    gr = gr + gfckv * dfck + gfc * dfc

    gdot = gdx * ux + gdy * uy + gdz * uz
    gx_ref[...] = (gdx - gdot * ux) * inv + gr * ux
    gy_ref[...] = (gdy - gdot * uy) * inv + gr * uy
    gz_ref[...] = (gdz - gdot * uz) * inv + gr * uz


def kernel(elems, descriptors, beta, energy, rij, unique_i, unique_j,
           tag_i, tag_j, W1, b1, W2, b2, W3, b3):
    f32 = jnp.float32
    P = rij.shape[0]
    K = _K
    N = P // K
    BN = _BN
    G = -(-N // BN)
    Npad = G * BN

    comp = rij.reshape(3, K, N)
    X = jnp.pad(comp[0], ((0, 0), (0, Npad - N)), constant_values=4.0)
    Y = jnp.pad(comp[1], ((0, 0), (0, Npad - N)))
    Z = jnp.pad(comp[2], ((0, 0), (0, Npad - N)))

    W1T = W1.T
    W2T = W2.T
    b1c = b1.reshape(-1, 1).astype(f32)
    b2c = b2.reshape(-1, 1).astype(f32)
    b3c = b3.reshape(1, 1).astype(f32)

    full = lambda a: pl.BlockSpec(a.shape, lambda i: (0,) * a.ndim)
    col = pl.BlockSpec((K, BN), lambda i: (0, i))

    eacc, GX, GY, GZ = pl.pallas_call(
        _pairnn_block,
        grid=(G,),
        in_specs=[col, col, col,
                  full(W1T), full(W1), full(W2T), full(W2),
                  full(W3), full(b1c), full(b2c), full(b3c)],
        out_specs=[pl.BlockSpec((8, BN), lambda i: (0, 0)),
                   col, col, col],
        out_shape=[jax.ShapeDtypeStruct((8, BN), f32),
                   jax.ShapeDtypeStruct((K, Npad), f32),
                   jax.ShapeDtypeStruct((K, Npad), f32),
                   jax.ShapeDtypeStruct((K, Npad), f32)],
    )(X, Y, Z, W1T, W1, W2T, W2, W3, b1c, b2c, b3c)

    total = jnp.sum(eacc)
    grad = jnp.stack([GX.reshape(-1)[:P], GY.reshape(-1)[:P],
                      GZ.reshape(-1)[:P]], axis=-1)
    return (total, grad)


# speedup vs baseline: 1.3526x; 1.3526x over previous
import functools

import jax
import jax.numpy as jnp
from jax.experimental import pallas as pl

_K = 16
_BN = 512
_CUTOFF = 3.0
_RMIN = 3.5
_ETA = 4.0
_NUM_RADIAL = 5
_NUM_3BODY = 12


def _pairnn_block(x_ref, y_ref, z_ref, w1t_ref, w1_ref, w2t_ref, w2_ref,
                  w3_ref, b1_ref, b2_ref, b3_ref,
                  eacc_ref, gx_ref, gy_ref, gz_ref):
    f32 = jnp.float32
    bf16 = jnp.bfloat16
    K, BN = _K, _BN
    BP = K * BN

    def b32(a):
        return a.astype(bf16).astype(f32)

    x = x_ref[...]
    y = y_ref[...]
    z = z_ref[...]

    r2 = x * x + y * y + z * z
    r = jnp.sqrt(r2)
    rm = jnp.maximum(r, 1e-12)
    inv = 1.0 / rm
    ux = x * inv
    uy = y * inv
    uz = z * inv

    aa = jnp.pi / (_CUTOFF - _RMIN)
    arg = aa * (r - _RMIN)
    in_tail = r > _RMIN
    fc = jnp.where(in_tail, 0.5 + 0.5 * jnp.cos(arg), 1.0)
    dfc = jnp.where(in_tail, -0.5 * aa * jnp.sin(arg), 0.0)

    bk = jnp.pi / _CUTOFF
    fck = 0.5 + 0.5 * jnp.cos(bk * r)
    dfck = -0.5 * bk * jnp.sin(bk * r)

    c0 = jnp.sqrt(2.0 / _CUTOFF)
    an = (bk * (1.0 + jnp.arange(_NUM_RADIAL, dtype=jnp.int32)
                .astype(f32))).reshape(_NUM_RADIAL, 1, 1)
    s5 = jnp.sin(an * r[None])
    c5 = jnp.cos(an * r[None])
    rbf = (c0 * fc * inv)[None] * s5

    uxb = b32(ux)
    uyb = b32(uy)
    uzb = b32(uz)
    eyemask = (1.0 - jnp.eye(K, dtype=f32))[:, :, None]
    gram = (uxb[:, None, :] * uxb[None, :, :] +
            uyb[:, None, :] * uyb[None, :, :] +
            uzb[:, None, :] * uzb[None, :, :]) * eyemask

    mu = (-1.0 + (2.0 / (_NUM_3BODY - 1)) *
          jnp.arange(_NUM_3BODY, dtype=jnp.int32).astype(f32))
    mu4 = mu.reshape(1, _NUM_3BODY, 1, 1)
    cm4 = jnp.exp(-_ETA * mu * mu).reshape(1, _NUM_3BODY, 1, 1)
    ga = (-_ETA) * gram * gram
    gb2 = (2.0 * _ETA) * gram
    e4 = jnp.exp(ga[:, None] + mu4 * gb2[:, None]) * cm4
    g3b = jnp.sum(e4 * fck[:, None, None, :], axis=0)

    desc = jnp.concatenate([rbf, g3b], axis=0).reshape(
        _NUM_RADIAL + _NUM_3BODY, BP)

    def dot(a, b):
        return jax.lax.dot_general(
            a.astype(bf16), b.astype(bf16),
            dimension_numbers=(((1,), (0,)), ((), ())),
            preferred_element_type=f32)

    h1p = dot(w1t_ref[...], desc) + b1_ref[...]
    h1 = jnp.maximum(h1p, 0.0)
    h2p = dot(w2t_ref[...], h1) + b2_ref[...]
    h2 = jnp.maximum(h2p, 0.0)
    pre3 = (jnp.sum(b32(h2) * b32(w3_ref[...]), axis=0, keepdims=True)
            + b3_ref[...])
    pre3 = pre3.reshape(K, BN)

    eij = pre3 * fc
    part = eij[:8, :] + eij[8:, :]

    @pl.when(pl.program_id(0) == 0)
    def _():
        eacc_ref[...] = jnp.zeros_like(eacc_ref)

    eacc_ref[...] += part

    gfc = pre3
    gpre3 = b32(fc).reshape(1, BP)
    gh2 = b32(w3_ref[...]) * gpre3
    gh2p = jnp.where(h2p > 0.0, gh2, 0.0)
    gh1 = dot(w2_ref[...], gh2p)
    gh1p = jnp.where(h1p > 0.0, gh1, 0.0)
    gdesc = dot(w1_ref[...], gh1p).reshape(
        _NUM_RADIAL + _NUM_3BODY, K, BN)

    grbf = gdesc[:_NUM_RADIAL]
    gd3b = gdesc[_NUM_RADIAL:]

    gfc = gfc + c0 * jnp.sum(grbf * s5, axis=0) * inv
    gr = (c0 * fc * inv) * jnp.sum(grbf * (an * c5 - s5 * inv[None]), axis=0)

    s0 = jnp.sum(e4 * gd3b[None], axis=1)
    s1 = jnp.sum(e4 * (gd3b * mu.reshape(-1, 1, 1))[None], axis=1)
    gfckv = jnp.sum(s0, axis=1)
    ggram = (-2.0 * _ETA) * (gram * s0 - s1) * \
        fck[:, None, :] * eyemask

    ggb = b32(ggram)
    gdx = (jnp.sum(ggb * uxb[:, None, :], axis=0) +
           jnp.sum(ggb * uxb[None, :, :], axis=1))
    gdy = (jnp.sum(ggb * uyb[:, None, :], axis=0) +
           jnp.sum(ggb * uyb[None, :, :], axis=1))
    gdz = (jnp.sum(ggb * uzb[:, None, :], axis=0) +
           jnp.sum(ggb * uzb[None, :, :], axis=1))

    gr = gr + gfckv * dfck + gfc * dfc

    gdot = gdx * ux + gdy * uy + gdz * uz
    gx_ref[...] = (gdx - gdot * ux) * inv + gr * ux
    gy_ref[...] = (gdy - gdot * uy) * inv + gr * uy
    gz_ref[...] = (gdz - gdot * uz) * inv + gr * uz


def kernel(elems, descriptors, beta, energy, rij, unique_i, unique_j,
           tag_i, tag_j, W1, b1, W2, b2, W3, b3):
    f32 = jnp.float32
    P = rij.shape[0]
    K = _K
    N = P // K
    BN = _BN
    G = -(-N // BN)
    Npad = G * BN

    comp = jnp.transpose(rij.reshape(N, K, 3), (2, 1, 0))
    X = jnp.pad(comp[0], ((0, 0), (0, Npad - N)), constant_values=4.0)
    Y = jnp.pad(comp[1], ((0, 0), (0, Npad - N)))
    Z = jnp.pad(comp[2], ((0, 0), (0, Npad - N)))

    W1T = W1.T
    W2T = W2.T
    b1c = b1.reshape(-1, 1).astype(f32)
    b2c = b2.reshape(-1, 1).astype(f32)
    b3c = b3.reshape(1, 1).astype(f32)

    full = lambda a: pl.BlockSpec(a.shape, lambda i: (0,) * a.ndim)
    col = pl.BlockSpec((K, BN), lambda i: (0, i))

    eacc, GX, GY, GZ = pl.pallas_call(
        _pairnn_block,
        grid=(G,),
        in_specs=[col, col, col,
                  full(W1T), full(W1), full(W2T), full(W2),
                  full(W3), full(b1c), full(b2c), full(b3c)],
        out_specs=[pl.BlockSpec((8, BN), lambda i: (0, 0)),
                   col, col, col],
        out_shape=[jax.ShapeDtypeStruct((8, BN), f32),
                   jax.ShapeDtypeStruct((K, Npad), f32),
                   jax.ShapeDtypeStruct((K, Npad), f32),
                   jax.ShapeDtypeStruct((K, Npad), f32)],
    )(X, Y, Z, W1T, W1, W2T, W2, W3, b1c, b2c, b3c)

    total = jnp.sum(eacc)
    grad = jnp.stack([GX[:, :N], GY[:, :N], GZ[:, :N]], axis=-1)
    grad = jnp.transpose(grad, (1, 0, 2)).reshape(P, 3)
    return (total, grad)


# speedup vs baseline: 1.4267x; 1.0548x over previous
import functools

import jax
import jax.numpy as jnp
from jax.experimental import pallas as pl

_K = 16
_BN = 512
_CUTOFF = 3.0
_RMIN = 3.5
_ETA = 4.0
_NUM_RADIAL = 5
_NUM_3BODY = 12


def _pairnn_block(x_ref, y_ref, z_ref, w1t_ref, w1_ref, w2t_ref, w2_ref,
                  w3_ref, b1_ref, b2_ref, b3_ref,
                  eacc_ref, gx_ref, gy_ref, gz_ref):
    f32 = jnp.float32
    bf16 = jnp.bfloat16
    K, BN = _K, _BN
    BP = K * BN

    def b32(a):
        return a.astype(bf16).astype(f32)

    x = x_ref[...]
    y = y_ref[...]
    z = z_ref[...]

    r2 = x * x + y * y + z * z
    r = jnp.sqrt(r2)
    rm = jnp.maximum(r, 1e-12)
    inv = 1.0 / rm
    ux = x * inv
    uy = y * inv
    uz = z * inv

    aa = jnp.pi / (_CUTOFF - _RMIN)
    arg = aa * (r - _RMIN)
    in_tail = r > _RMIN
    fc = jnp.where(in_tail, 0.5 + 0.5 * jnp.cos(arg), 1.0)
    dfc = jnp.where(in_tail, -0.5 * aa * jnp.sin(arg), 0.0)

    bk = jnp.pi / _CUTOFF
    s1 = jnp.sin(bk * r)
    c1 = jnp.cos(bk * r)
    fck = 0.5 + 0.5 * c1
    dfck = -0.5 * bk * s1

    c0 = jnp.sqrt(2.0 / _CUTOFF)
    an = (bk * (1.0 + jnp.arange(_NUM_RADIAL, dtype=jnp.int32)
                .astype(f32))).reshape(_NUM_RADIAL, 1, 1)
    twoc = 2.0 * c1
    s2 = twoc * s1
    c2 = twoc * c1 - 1.0
    s3 = twoc * s2 - s1
    c3 = twoc * c2 - c1
    s4 = twoc * s3 - s2
    c4 = twoc * c3 - c2
    s5n = twoc * s4 - s3
    c5n = twoc * c4 - c3
    s5 = jnp.stack([s1, s2, s3, s4, s5n])
    c5 = jnp.stack([c1, c2, c3, c4, c5n])
    rbf = (c0 * fc * inv)[None] * s5

    uxb = b32(ux)
    uyb = b32(uy)
    uzb = b32(uz)
    eyemask = (1.0 - jnp.eye(K, dtype=f32))[:, :, None]
    gram = (uxb[:, None, :] * uxb[None, :, :] +
            uyb[:, None, :] * uyb[None, :, :] +
            uzb[:, None, :] * uzb[None, :, :]) * eyemask

    mu = (-1.0 + (2.0 / (_NUM_3BODY - 1)) *
          jnp.arange(_NUM_3BODY, dtype=jnp.int32).astype(f32))
    mu4 = mu.reshape(1, _NUM_3BODY, 1, 1)
    cm4 = jnp.exp(-_ETA * mu * mu).reshape(1, _NUM_3BODY, 1, 1)
    ga = (-_ETA) * gram * gram
    gb2 = (2.0 * _ETA) * gram
    e4 = jnp.exp(ga[:, None] + mu4 * gb2[:, None]) * cm4
    g3b = jnp.sum(e4 * fck[:, None, None, :], axis=0)

    desc = jnp.concatenate([rbf, g3b], axis=0).reshape(
        _NUM_RADIAL + _NUM_3BODY, BP)

    def dot(a, b):
        return jax.lax.dot_general(
            a.astype(bf16), b.astype(bf16),
            dimension_numbers=(((1,), (0,)), ((), ())),
            preferred_element_type=f32)

    h1p = dot(w1t_ref[...], desc) + b1_ref[...]
    h1 = jnp.maximum(h1p, 0.0)
    h2p = dot(w2t_ref[...], h1) + b2_ref[...]
    h2 = jnp.maximum(h2p, 0.0)
    pre3 = (jnp.sum(b32(h2) * b32(w3_ref[...]), axis=0, keepdims=True)
            + b3_ref[...])
    pre3 = pre3.reshape(K, BN)

    eij = pre3 * fc
    part = eij[:8, :] + eij[8:, :]

    @pl.when(pl.program_id(0) == 0)
    def _():
        eacc_ref[...] = jnp.zeros_like(eacc_ref)

    eacc_ref[...] += part

    gfc = pre3
    gpre3 = b32(fc).reshape(1, BP)
    gh2 = b32(w3_ref[...]) * gpre3
    gh2p = jnp.where(h2p > 0.0, gh2, 0.0)
    gh1 = dot(w2_ref[...], gh2p)
    gh1p = jnp.where(h1p > 0.0, gh1, 0.0)
    gdesc = dot(w1_ref[...], gh1p).reshape(
        _NUM_RADIAL + _NUM_3BODY, K, BN)

    grbf = gdesc[:_NUM_RADIAL]
    gd3b = gdesc[_NUM_RADIAL:]

    gfc = gfc + c0 * jnp.sum(grbf * s5, axis=0) * inv
    gr = (c0 * fc * inv) * jnp.sum(grbf * (an * c5 - s5 * inv[None]), axis=0)

    s0 = jnp.sum(e4 * gd3b[None], axis=1)
    s1 = jnp.sum(e4 * (gd3b * mu.reshape(-1, 1, 1))[None], axis=1)
    gfckv = jnp.sum(s0, axis=1)
    ggram = (-2.0 * _ETA) * (gram * s0 - s1) * \
        fck[:, None, :] * eyemask

    ggb = b32(ggram)
    gdx = (jnp.sum(ggb * uxb[:, None, :], axis=0) +
           jnp.sum(ggb * uxb[None, :, :], axis=1))
    gdy = (jnp.sum(ggb * uyb[:, None, :], axis=0) +
           jnp.sum(ggb * uyb[None, :, :], axis=1))
    gdz = (jnp.sum(ggb * uzb[:, None, :], axis=0) +
           jnp.sum(ggb * uzb[None, :, :], axis=1))

    gr = gr + gfckv * dfck + gfc * dfc

    gdot = gdx * ux + gdy * uy + gdz * uz
    gx_ref[...] = (gdx - gdot * ux) * inv + gr * ux
    gy_ref[...] = (gdy - gdot * uy) * inv + gr * uy
    gz_ref[...] = (gdz - gdot * uz) * inv + gr * uz


def kernel(elems, descriptors, beta, energy, rij, unique_i, unique_j,
           tag_i, tag_j, W1, b1, W2, b2, W3, b3):
    f32 = jnp.float32
    P = rij.shape[0]
    K = _K
    N = P // K
    BN = _BN
    G = -(-N // BN)
    Npad = G * BN

    comp = jnp.transpose(rij.reshape(N, K, 3), (2, 1, 0))
    X = jnp.pad(comp[0], ((0, 0), (0, Npad - N)), constant_values=4.0)
    Y = jnp.pad(comp[1], ((0, 0), (0, Npad - N)))
    Z = jnp.pad(comp[2], ((0, 0), (0, Npad - N)))

    W1T = W1.T
    W2T = W2.T
    b1c = b1.reshape(-1, 1).astype(f32)
    b2c = b2.reshape(-1, 1).astype(f32)
    b3c = b3.reshape(1, 1).astype(f32)

    full = lambda a: pl.BlockSpec(a.shape, lambda i: (0,) * a.ndim)
    col = pl.BlockSpec((K, BN), lambda i: (0, i))

    eacc, GX, GY, GZ = pl.pallas_call(
        _pairnn_block,
        grid=(G,),
        in_specs=[col, col, col,
                  full(W1T), full(W1), full(W2T), full(W2),
                  full(W3), full(b1c), full(b2c), full(b3c)],
        out_specs=[pl.BlockSpec((8, BN), lambda i: (0, 0)),
                   col, col, col],
        out_shape=[jax.ShapeDtypeStruct((8, BN), f32),
                   jax.ShapeDtypeStruct((K, Npad), f32),
                   jax.ShapeDtypeStruct((K, Npad), f32),
                   jax.ShapeDtypeStruct((K, Npad), f32)],
    )(X, Y, Z, W1T, W1, W2T, W2, W3, b1c, b2c, b3c)

    total = jnp.sum(eacc)
    grad = jnp.stack([GX[:, :N], GY[:, :N], GZ[:, :N]], axis=-1)
    grad = jnp.transpose(grad, (1, 0, 2)).reshape(P, 3)
    return (total, grad)
